# speedup vs baseline: 83.3672x; 83.3672x over previous
import functools

import jax
import jax.numpy as jnp
from jax.experimental import pallas as pl
from jax.experimental.pallas import tpu as pltpu

_HIGH = jax.lax.Precision.HIGHEST


def _dot(a, b):
    return jax.lax.dot_general(a, b, (((1,), (0,)), ((), ())),
                               precision=_HIGH, preferred_element_type=jnp.float32)


def _prep_body(xT_ref, Wsrc_ref, Wtgt_ref, b_ref, adj_ref, PT_ref, QT_ref, adjf_ref):
    xT = xT_ref[...]
    PT_ref[...] = _dot(Wsrc_ref[...], xT)
    QT_ref[...] = _dot(Wtgt_ref[...], xT) + b_ref[...]
    adjf_ref[...] = (adj_ref[...] != 0).astype(jnp.float32)


def _msg_body(we_ref, pcols_ref, qT_ref, e_ref, adjf_ref, out_ref, *, db):
    i = pl.program_id(0)
    e = e_ref[...]
    a = adjf_ref[...]
    for j in range(db):
        w = we_ref[0, i * db + j]
        arg = e * w + (pcols_ref[j] + qT_ref[j:j + 1, :])
        v = jnp.maximum(arg, 0.0) * a
        out_ref[j, :] = jnp.sum(v, axis=0)


def _gru_body(aggT_ref, xT_ref, Wih_ref, Whh_ref, bih_ref, bhh_ref,
              Wpol_ref, bpol_ref, Wval_ref, bval_ref, q_ref, v_ref):
    d = aggT_ref.shape[0]
    xT = xT_ref[...]
    giT = _dot(Wih_ref[...], aggT_ref[...]) + bih_ref[...]
    ghT = _dot(Whh_ref[...], xT) + bhh_ref[...]
    r = jax.nn.sigmoid(giT[0:d, :] + ghT[0:d, :])
    z = jax.nn.sigmoid(giT[d:2 * d, :] + ghT[d:2 * d, :])
    n = jnp.tanh(giT[2 * d:3 * d, :] + r * ghT[2 * d:3 * d, :])
    updT = (1.0 - z) * n + z * xT
    s = jnp.sum(updT, axis=1, keepdims=True)
    q_ref[...] = _dot(Wpol_ref[...], s) + bpol_ref[...]
    v_ref[...] = _dot(Wval_ref[...], s) + bval_ref[...]


def kernel(node_features, edge_features, adjacency_matrix,
           W_msg, b_msg, W_ih, W_hh, b_ih, b_hh, W_pol, b_pol, W_val, b_val):
    N, D = node_features.shape
    A = W_pol.shape[0]
    DB = 8

    xT = node_features.T
    Wsrc = W_msg[:, :D]
    Wtgt = W_msg[:, D:2 * D]
    w_e = W_msg[:, 2 * D].reshape(1, D)

    PT, QT, adjf = pl.pallas_call(
        _prep_body,
        out_shape=(
            jax.ShapeDtypeStruct((D, N), jnp.float32),
            jax.ShapeDtypeStruct((D, N), jnp.float32),
            jax.ShapeDtypeStruct((N, N), jnp.float32),
        ),
    )(xT, Wsrc, Wtgt, b_msg.reshape(D, 1), adjacency_matrix)

    Pcols = PT.reshape(D, N, 1)

    aggT = pl.pallas_call(
        functools.partial(_msg_body, db=DB),
        grid=(D // DB,),
        in_specs=[
            pl.BlockSpec(memory_space=pltpu.SMEM),
            pl.BlockSpec((DB, N, 1), lambda i: (i, 0, 0)),
            pl.BlockSpec((DB, N), lambda i: (i, 0)),
            pl.BlockSpec((N, N), lambda i: (0, 0)),
            pl.BlockSpec((N, N), lambda i: (0, 0)),
        ],
        out_specs=pl.BlockSpec((DB, N), lambda i: (i, 0)),
        out_shape=jax.ShapeDtypeStruct((D, N), jnp.float32),
    )(w_e, Pcols, QT, edge_features, adjf)

    q, v = pl.pallas_call(
        _gru_body,
        out_shape=(
            jax.ShapeDtypeStruct((A, 1), jnp.float32),
            jax.ShapeDtypeStruct((1, 1), jnp.float32),
        ),
    )(aggT, xT, W_ih, W_hh, b_ih.reshape(3 * D, 1), b_hh.reshape(3 * D, 1),
      W_pol, b_pol.reshape(A, 1), W_val, b_val.reshape(1, 1))

    return q.reshape(A), v.reshape(1)


# speedup vs baseline: 105.5119x; 1.2656x over previous
import functools

import jax
import jax.numpy as jnp
from jax.experimental import pallas as pl
from jax.experimental.pallas import tpu as pltpu


def _bdot(a, b):
    return jax.lax.dot_general(a.astype(jnp.bfloat16), b.astype(jnp.bfloat16),
                               (((1,), (0,)), ((), ())),
                               preferred_element_type=jnp.float32)


def _body(we_ref, x_ref, xT_ref, wsT_ref, wtgt_ref, bcol_ref, adj_ref, e_ref,
          Wih_ref, Whh_ref, bih_ref, bhh_ref, Wpol_ref, bpol_ref, Wval_ref, bval_ref,
          q_ref, v_ref, adjf_s, er_s, aggT_s, *, db, nsteps):
    i = pl.program_id(0)

    @pl.when(i == 0)
    def _prep():
        adjf_s[...] = (adj_ref[...] != 0).astype(jnp.float32)
        er_s[...] = e_ref[...].astype(jnp.bfloat16).astype(jnp.float32)

    pblk = _bdot(x_ref[...], wsT_ref[0])
    qblk = _bdot(wtgt_ref[...], xT_ref[...]) + bcol_ref[...]
    a = adjf_s[...]
    ev = er_s[...]
    for j in range(db):
        w = we_ref[0, i * db + j]
        arg = ev * w + (pblk[:, j:j + 1] + qblk[j:j + 1, :])
        msk = jnp.maximum(arg, 0.0) * a
        aggT_s[pl.ds(i * db + j, 1), :] = jnp.sum(msk, axis=0, keepdims=True)

    @pl.when(i == nsteps - 1)
    def _gru():
        d = aggT_s.shape[0]
        xT = xT_ref[...]
        giT = _bdot(Wih_ref[...], aggT_s[...]) + bih_ref[...]
        ghT = _bdot(Whh_ref[...], xT) + bhh_ref[...]
        r = jax.nn.sigmoid(giT[0:d, :] + ghT[0:d, :])
        z = jax.nn.sigmoid(giT[d:2 * d, :] + ghT[d:2 * d, :])
        n = jnp.tanh(giT[2 * d:3 * d, :] + r * ghT[2 * d:3 * d, :])
        updT = (1.0 - z) * n + z * xT
        s = jnp.sum(updT, axis=1, keepdims=True)
        q_ref[...] = _bdot(Wpol_ref[...], s) + bpol_ref[...]
        v_ref[...] = jax.lax.dot_general(
            Wval_ref[...], s, (((1,), (0,)), ((), ())),
            precision=jax.lax.Precision.HIGHEST,
            preferred_element_type=jnp.float32) + bval_ref[...]


def kernel(node_features, edge_features, adjacency_matrix,
           W_msg, b_msg, W_ih, W_hh, b_ih, b_hh, W_pol, b_pol, W_val, b_val):
    N, D = node_features.shape
    A = W_pol.shape[0]
    DB = 8
    NSTEPS = D // DB

    xT = node_features.T
    wsT = W_msg[:, :D].T.reshape(D, NSTEPS, DB).transpose(1, 0, 2)
    wtgt = W_msg[:, D:2 * D]
    w_e = W_msg[:, 2 * D].astype(jnp.bfloat16).astype(jnp.float32).reshape(1, D)

    const = lambda *bs: pl.BlockSpec(bs, lambda i: (0,) * len(bs))
    q, v = pl.pallas_call(
        functools.partial(_body, db=DB, nsteps=NSTEPS),
        grid=(NSTEPS,),
        in_specs=[
            pl.BlockSpec(memory_space=pltpu.SMEM),
            const(N, D),
            const(D, N),
            pl.BlockSpec((1, D, DB), lambda i: (i, 0, 0)),
            pl.BlockSpec((DB, D), lambda i: (i, 0)),
            pl.BlockSpec((DB, 1), lambda i: (i, 0)),
            const(N, N),
            const(N, N),
            const(3 * D, D), const(3 * D, D),
            const(3 * D, 1), const(3 * D, 1),
            const(A, D), const(A, 1),
            const(1, D), const(1, 1),
        ],
        out_specs=(const(A, 1), const(1, 1)),
        out_shape=(
            jax.ShapeDtypeStruct((A, 1), jnp.float32),
            jax.ShapeDtypeStruct((1, 1), jnp.float32),
        ),
        scratch_shapes=[
            pltpu.VMEM((N, N), jnp.float32),
            pltpu.VMEM((N, N), jnp.float32),
            pltpu.VMEM((D, N), jnp.float32),
        ],
    )(w_e, node_features, xT, wsT, wtgt, b_msg.reshape(D, 1), adjacency_matrix,
      edge_features, W_ih, W_hh, b_ih.reshape(3 * D, 1), b_hh.reshape(3 * D, 1),
      W_pol, b_pol.reshape(A, 1), W_val, b_val.reshape(1, 1))

    return q.reshape(A), v.reshape(1)


# speedup vs baseline: 106.2243x; 1.0068x over previous
import functools

import jax
import jax.numpy as jnp
from jax.experimental import pallas as pl
from jax.experimental.pallas import tpu as pltpu


def _mxu(a, b):
    return jax.lax.dot_general(a, b, (((1,), (0,)), ((), ())),
                               preferred_element_type=jnp.float32)


def _bdot(a, b):
    return _mxu(a.astype(jnp.bfloat16), b.astype(jnp.bfloat16))


def _body(we_ref, xbf_ref, xTbf_ref, xT_ref, wsT_ref, wtgt_ref, bcol_ref, adj_ref, e_ref,
          Wih_ref, Whh_ref, bih_ref, bhh_ref, Wpol_ref, bpol_ref, Wval_ref, bval_ref,
          q_ref, v_ref, adjf_s, er_s, aggT_s, *, db, nsteps):
    i = pl.program_id(0)

    @pl.when(i == 0)
    def _prep():
        adjf_s[...] = (adj_ref[...] != 0).astype(jnp.float32)
        er_s[...] = e_ref[...].astype(jnp.bfloat16).astype(jnp.float32)

    pblk = _mxu(xbf_ref[...], wsT_ref[0].astype(jnp.bfloat16))
    qblk = _mxu(wtgt_ref[...].astype(jnp.bfloat16), xTbf_ref[...]) + bcol_ref[...]
    a = adjf_s[...]
    ev = er_s[...]
    for j in range(db):
        w = we_ref[0, i * db + j]
        arg = ev * w + (pblk[:, j:j + 1] + qblk[j:j + 1, :])
        msk = jnp.maximum(arg, 0.0) * a
        aggT_s[pl.ds(i * db + j, 1), :] = jnp.sum(msk, axis=0, keepdims=True)

    @pl.when(i == nsteps - 1)
    def _gru():
        d = aggT_s.shape[0]
        xT = xT_ref[...]
        giT = _bdot(Wih_ref[...], aggT_s[...]) + bih_ref[...]
        ghT = _mxu(Whh_ref[...].astype(jnp.bfloat16), xTbf_ref[...]) + bhh_ref[...]
        r = jax.nn.sigmoid(giT[0:d, :] + ghT[0:d, :])
        z = jax.nn.sigmoid(giT[d:2 * d, :] + ghT[d:2 * d, :])
        n = jnp.tanh(giT[2 * d:3 * d, :] + r * ghT[2 * d:3 * d, :])
        updT = (1.0 - z) * n + z * xT
        s = jnp.sum(updT, axis=1, keepdims=True)
        q_ref[...] = _bdot(Wpol_ref[...], s) + bpol_ref[...]
        v_ref[...] = jax.lax.dot_general(
            Wval_ref[...], s, (((1,), (0,)), ((), ())),
            precision=jax.lax.Precision.HIGHEST,
            preferred_element_type=jnp.float32) + bval_ref[...]


def kernel(node_features, edge_features, adjacency_matrix,
           W_msg, b_msg, W_ih, W_hh, b_ih, b_hh, W_pol, b_pol, W_val, b_val):
    N, D = node_features.shape
    A = W_pol.shape[0]
    DB = 16
    NSTEPS = D // DB

    xT = node_features.T
    wsT = W_msg[:, :D].T.reshape(D, NSTEPS, DB).transpose(1, 0, 2)
    wtgt = W_msg[:, D:2 * D]
    w_e = W_msg[:, 2 * D].astype(jnp.bfloat16).astype(jnp.float32).reshape(1, D)

    const = lambda *bs: pl.BlockSpec(bs, lambda i: (0,) * len(bs))
    q, v = pl.pallas_call(
        functools.partial(_body, db=DB, nsteps=NSTEPS),
        grid=(NSTEPS,),
        in_specs=[
            pl.BlockSpec(memory_space=pltpu.SMEM),
            const(N, D),
            const(D, N),
            const(D, N),
            pl.BlockSpec((1, D, DB), lambda i: (i, 0, 0)),
            pl.BlockSpec((DB, D), lambda i: (i, 0)),
            pl.BlockSpec((DB, 1), lambda i: (i, 0)),
            const(N, N),
            const(N, N),
            const(3 * D, D), const(3 * D, D),
            const(3 * D, 1), const(3 * D, 1),
            const(A, D), const(A, 1),
            const(1, D), const(1, 1),
        ],
        out_specs=(const(A, 1), const(1, 1)),
        out_shape=(
            jax.ShapeDtypeStruct((A, 1), jnp.float32),
            jax.ShapeDtypeStruct((1, 1), jnp.float32),
        ),
        scratch_shapes=[
            pltpu.VMEM((N, N), jnp.float32),
            pltpu.VMEM((N, N), jnp.float32),
            pltpu.VMEM((D, N), jnp.float32),
        ],
    )(w_e, node_features.astype(jnp.bfloat16), xT.astype(jnp.bfloat16), xT,
      wsT, wtgt, b_msg.reshape(D, 1), adjacency_matrix,
      edge_features, W_ih, W_hh, b_ih.reshape(3 * D, 1), b_hh.reshape(3 * D, 1),
      W_pol, b_pol.reshape(A, 1), W_val, b_val.reshape(1, 1))

    return q.reshape(A), v.reshape(1)
